# Initial kernel scaffold; baseline (speedup 1.0000x reference)
import jax
import jax.numpy as jnp
from jax.experimental import pallas as pl


def kernel(x, edge_index, W1, b1, W2, b2):
    raise NotImplementedError("write your pallas kernel here")


# speedup vs baseline: 26.3908x; 26.3908x over previous
import jax
import jax.numpy as jnp
from jax import lax
from jax.experimental import pallas as pl
from jax.experimental.pallas import tpu as pltpu
from jax.experimental.pallas import tpu_sc as plsc

N_NODES = 100000
N_EDGES = 6400000
NP = 100352
NC, NS = 2, 16
NW = NC * NS
BLK = 128
NB = 1562
MAIN = NW * NB * BLK
ROWS_PT = NP // NS

_MESH = plsc.VectorSubcoreMesh(
    core_axis_name="c", subcore_axis_name="s", num_cores=NC, num_subcores=NS)

F32 = jnp.float32
I32 = jnp.int32


def _writeback(acc_sh, stage, out0, out1, c, s):
    plsc.subcore_barrier()
    pltpu.sync_copy(acc_sh.at[pl.ds(s * ROWS_PT, ROWS_PT)], stage)

    @pl.when(c == 0)
    def _():
        pltpu.sync_copy(stage, out0.at[pl.ds(s * ROWS_PT, ROWS_PT)])

    @pl.when(c == 1)
    def _():
        pltpu.sync_copy(stage, out1.at[pl.ds(s * ROWS_PT, ROWS_PT)])


def _deg_body(dsts, zeros_h, out0, out1, deg_sh, dst_v, ones_v, stage):
    c = lax.axis_index("c")
    s = lax.axis_index("s")
    wid = c * NS + s
    one = jnp.ones((16,), F32)
    for i in range(BLK // 16):
        ones_v[pl.ds(i * 16, 16)] = one
    pltpu.sync_copy(zeros_h.at[pl.ds(s * ROWS_PT, ROWS_PT)], stage)
    pltpu.sync_copy(stage, deg_sh.at[pl.ds(s * ROWS_PT, ROWS_PT)])
    plsc.subcore_barrier()

    def body(i, carry):
        base = wid * (NB * BLK) + i * BLK
        pltpu.sync_copy(dsts.at[pl.ds(base, BLK)], dst_v)
        pltpu.sync_copy(ones_v, deg_sh.at[dst_v], add=True)
        return carry

    lax.fori_loop(0, NB, body, 0)

    @pl.when(wid < 16)
    def _():
        base = MAIN + wid * BLK
        pltpu.sync_copy(dsts.at[pl.ds(base, BLK)], dst_v)
        pltpu.sync_copy(ones_v, deg_sh.at[dst_v], add=True)

    _writeback(deg_sh, stage, out0, out1, c, s)


_deg_call = pl.kernel(
    _deg_body,
    out_type=(jax.ShapeDtypeStruct((NP,), F32),
              jax.ShapeDtypeStruct((NP,), F32)),
    mesh=_MESH,
    scratch_types=[
        pltpu.VMEM_SHARED((NP,), F32),
        pltpu.VMEM((BLK,), I32),
        pltpu.VMEM((BLK,), F32),
        pltpu.VMEM((ROWS_PT,), F32),
    ],
)


def _s1_body(srcs, dsts, g1_h, zeros_h, out0, out1, acc_sh, src_v, dst_v,
             rows_v, stage):
    c = lax.axis_index("c")
    s = lax.axis_index("s")
    wid = c * NS + s
    pltpu.sync_copy(zeros_h.at[pl.ds(s * ROWS_PT, ROWS_PT)], stage)
    pltpu.sync_copy(stage, acc_sh.at[pl.ds(s * ROWS_PT, ROWS_PT)])
    plsc.subcore_barrier()

    def body(i, carry):
        base = wid * (NB * BLK) + i * BLK
        pltpu.sync_copy(srcs.at[pl.ds(base, BLK)], src_v)
        pltpu.sync_copy(dsts.at[pl.ds(base, BLK)], dst_v)
        pltpu.sync_copy(g1_h.at[src_v], rows_v)
        pltpu.sync_copy(rows_v, acc_sh.at[dst_v], add=True)
        return carry

    lax.fori_loop(0, NB, body, 0)

    @pl.when(wid < 16)
    def _():
        base = MAIN + wid * BLK
        pltpu.sync_copy(srcs.at[pl.ds(base, BLK)], src_v)
        pltpu.sync_copy(dsts.at[pl.ds(base, BLK)], dst_v)
        pltpu.sync_copy(g1_h.at[src_v], rows_v)
        pltpu.sync_copy(rows_v, acc_sh.at[dst_v], add=True)

    _writeback(acc_sh, stage, out0, out1, c, s)


_s1_call = pl.kernel(
    _s1_body,
    out_type=(jax.ShapeDtypeStruct((NP, 8), F32),
              jax.ShapeDtypeStruct((NP, 8), F32)),
    mesh=_MESH,
    scratch_types=[
        pltpu.VMEM_SHARED((NP, 8), F32),
        pltpu.VMEM((BLK,), I32),
        pltpu.VMEM((BLK,), I32),
        pltpu.VMEM((BLK, 8), F32),
        pltpu.VMEM((ROWS_PT, 8), F32),
    ],
    compiler_params=pltpu.CompilerParams(use_tc_tiling_on_sc=False),
)


def _s2_body(srcs, dsts, z_h, zeros_h, out0, out1, acc_sh, src_v, dst_v,
             val_v, stage):
    c = lax.axis_index("c")
    s = lax.axis_index("s")
    wid = c * NS + s
    pltpu.sync_copy(zeros_h.at[pl.ds(s * ROWS_PT, ROWS_PT)], stage)
    pltpu.sync_copy(stage, acc_sh.at[pl.ds(s * ROWS_PT, ROWS_PT)])
    plsc.subcore_barrier()

    def body(i, carry):
        base = wid * (NB * BLK) + i * BLK
        pltpu.sync_copy(srcs.at[pl.ds(base, BLK)], src_v)
        pltpu.sync_copy(dsts.at[pl.ds(base, BLK)], dst_v)
        pltpu.sync_copy(z_h.at[src_v], val_v)
        pltpu.sync_copy(val_v, acc_sh.at[dst_v], add=True)
        return carry

    lax.fori_loop(0, NB, body, 0)

    @pl.when(wid < 16)
    def _():
        base = MAIN + wid * BLK
        pltpu.sync_copy(srcs.at[pl.ds(base, BLK)], src_v)
        pltpu.sync_copy(dsts.at[pl.ds(base, BLK)], dst_v)
        pltpu.sync_copy(z_h.at[src_v], val_v)
        pltpu.sync_copy(val_v, acc_sh.at[dst_v], add=True)

    _writeback(acc_sh, stage, out0, out1, c, s)


_s2_call = pl.kernel(
    _s2_body,
    out_type=(jax.ShapeDtypeStruct((NP,), F32),
              jax.ShapeDtypeStruct((NP,), F32)),
    mesh=_MESH,
    scratch_types=[
        pltpu.VMEM_SHARED((NP,), F32),
        pltpu.VMEM((BLK,), I32),
        pltpu.VMEM((BLK,), I32),
        pltpu.VMEM((BLK,), F32),
        pltpu.VMEM((ROWS_PT,), F32),
    ],
)


_R = 512
_GRID = NP // _R


def _tc1_body(deg0_ref, deg1_ref, x_ref, dinv_ref, g1_ref):
    deg = deg0_ref[...] + deg1_ref[...] + 1.0
    dinv = lax.rsqrt(deg)
    dinv_ref[...] = dinv
    g1_ref[...] = jnp.concatenate(
        [x_ref[...] * dinv, jnp.zeros((_R, 3), F32)], axis=1)


def _tc1_call(deg0, deg1, x_pad):
    return pl.pallas_call(
        _tc1_body,
        grid=(_GRID,),
        in_specs=[
            pl.BlockSpec((_R, 1), lambda i: (i, 0)),
            pl.BlockSpec((_R, 1), lambda i: (i, 0)),
            pl.BlockSpec((_R, 5), lambda i: (i, 0)),
        ],
        out_specs=[
            pl.BlockSpec((_R, 1), lambda i: (i, 0)),
            pl.BlockSpec((_R, 8), lambda i: (i, 0)),
        ],
        out_shape=[
            jax.ShapeDtypeStruct((NP, 1), F32),
            jax.ShapeDtypeStruct((NP, 8), F32),
        ],
    )(deg0, deg1, x_pad)


def _tc2_body(s1p0_ref, s1p1_ref, x_ref, dinv_ref, w1_ref, b1_ref, w2_ref,
              z_ref):
    s1 = s1p0_ref[...] + s1p1_ref[...]
    dinv = dinv_ref[...]
    u = dinv * s1[:, :5] + (dinv * dinv) * x_ref[...]
    h = jnp.maximum(
        jnp.dot(u, w1_ref[...], preferred_element_type=F32) + b1_ref[...],
        0.0)
    z_ref[...] = dinv * jnp.dot(h, w2_ref[...], preferred_element_type=F32)


def _tc2_call(s1p0, s1p1, x_pad, dinv, W1, b1r, W2):
    return pl.pallas_call(
        _tc2_body,
        grid=(_GRID,),
        in_specs=[
            pl.BlockSpec((_R, 8), lambda i: (i, 0)),
            pl.BlockSpec((_R, 8), lambda i: (i, 0)),
            pl.BlockSpec((_R, 5), lambda i: (i, 0)),
            pl.BlockSpec((_R, 1), lambda i: (i, 0)),
            pl.BlockSpec((5, 16), lambda i: (0, 0)),
            pl.BlockSpec((1, 16), lambda i: (0, 0)),
            pl.BlockSpec((16, 1), lambda i: (0, 0)),
        ],
        out_specs=pl.BlockSpec((_R, 1), lambda i: (i, 0)),
        out_shape=jax.ShapeDtypeStruct((NP, 1), F32),
    )(s1p0, s1p1, x_pad, dinv, W1, b1r, W2)


def _tc3_body(s2p0_ref, s2p1_ref, dinv_ref, z_ref, b2_ref, out_ref):
    dinv = dinv_ref[...]
    out_ref[...] = dinv * (s2p0_ref[...] + s2p1_ref[...]) \
        + dinv * z_ref[...] + b2_ref[...]


def _tc3_call(s2p0, s2p1, dinv, z, b2r):
    return pl.pallas_call(
        _tc3_body,
        grid=(_GRID,),
        in_specs=[
            pl.BlockSpec((_R, 1), lambda i: (i, 0)),
            pl.BlockSpec((_R, 1), lambda i: (i, 0)),
            pl.BlockSpec((_R, 1), lambda i: (i, 0)),
            pl.BlockSpec((_R, 1), lambda i: (i, 0)),
            pl.BlockSpec((1, 1), lambda i: (0, 0)),
        ],
        out_specs=pl.BlockSpec((_R, 1), lambda i: (i, 0)),
        out_shape=jax.ShapeDtypeStruct((NP, 1), F32),
    )(s2p0, s2p1, dinv, z, b2r)


def kernel(x, edge_index, W1, b1, W2, b2):
    srcs = edge_index[0]
    dsts = edge_index[1]
    x_pad = jnp.pad(x, ((0, NP - N_NODES), (0, 0)))

    zeros1 = jnp.zeros((NP,), F32)
    zeros8 = jnp.zeros((NP, 8), F32)

    deg0, deg1 = _deg_call(dsts, zeros1)
    dinv, g1 = _tc1_call(deg0.reshape(NP, 1),
                         deg1.reshape(NP, 1), x_pad)
    s1p0, s1p1 = _s1_call(srcs, dsts, g1, zeros8)
    z = _tc2_call(s1p0, s1p1, x_pad, dinv,
                  W1, b1.reshape(1, 16), W2)
    s2p0, s2p1 = _s2_call(srcs, dsts, z.reshape(NP), zeros1)
    out = _tc3_call(s2p0.reshape(NP, 1), s2p1.reshape(NP, 1),
                    dinv, z, b2.reshape(1, 1))
    return out[:N_NODES]


# speedup vs baseline: 107.7017x; 4.0810x over previous
import jax
import jax.numpy as jnp
from jax import lax
from jax.experimental import pallas as pl
from jax.experimental.pallas import tpu as pltpu
from jax.experimental.pallas import tpu_sc as plsc

N_NODES = 100000
N_EDGES = 6400000
NP = 100352
NC, NS = 2, 16
NW = NC * NS
BLK = 128
CB = 22
NCH = 71
NB = NCH * CB
MAIN = NW * NB * BLK
ROWS_PT = NP // NS

_MESH = plsc.VectorSubcoreMesh(
    core_axis_name="c", subcore_axis_name="s", num_cores=NC, num_subcores=NS)

F32 = jnp.float32
I32 = jnp.int32


def _writeback(acc_sh, stage, out0, out1, c, s):
    plsc.subcore_barrier()
    pltpu.sync_copy(acc_sh.at[pl.ds(s * ROWS_PT, ROWS_PT)], stage)

    @pl.when(c == 0)
    def _():
        pltpu.sync_copy(stage, out0.at[pl.ds(s * ROWS_PT, ROWS_PT)])

    @pl.when(c == 1)
    def _():
        pltpu.sync_copy(stage, out1.at[pl.ds(s * ROWS_PT, ROWS_PT)])


def _deg_body(dsts, zeros_h, out0, out1, *scr):
    deg_sh, ones_v, stage, sem_in, sem_s = scr[:5]
    dst_b = scr[5:5 + CB]
    c = lax.axis_index("c")
    s = lax.axis_index("s")
    wid = c * NS + s
    one = jnp.ones((16,), F32)
    for i in range(BLK // 16):
        ones_v[pl.ds(i * 16, 16)] = one
    pltpu.sync_copy(zeros_h.at[pl.ds(s * ROWS_PT, ROWS_PT)], stage)
    pltpu.sync_copy(stage, deg_sh.at[pl.ds(s * ROWS_PT, ROWS_PT)])
    plsc.subcore_barrier()

    def chunk(i, carry):
        @pl.when(i > 0)
        def _():
            for j in range(CB):
                pltpu.make_async_copy(ones_v, deg_sh.at[dst_b[j]],
                                      sem_s).wait()
        base = wid * (NB * BLK) + i * (CB * BLK)
        for j in range(CB):
            pltpu.async_copy(dsts.at[pl.ds(base + j * BLK, BLK)], dst_b[j],
                             sem_in)
        for j in range(CB):
            pltpu.make_async_copy(dsts.at[pl.ds(base + j * BLK, BLK)],
                                  dst_b[j], sem_in).wait()
        for j in range(CB):
            pltpu.async_copy(ones_v, deg_sh.at[dst_b[j]], sem_s, add=True)
        return carry

    lax.fori_loop(0, NCH, chunk, 0)
    for j in range(CB):
        pltpu.make_async_copy(ones_v, deg_sh.at[dst_b[j]], sem_s).wait()

    @pl.when(wid < 16)
    def _():
        base = MAIN + wid * BLK
        pltpu.sync_copy(dsts.at[pl.ds(base, BLK)], dst_b[0])
        pltpu.sync_copy(ones_v, deg_sh.at[dst_b[0]], add=True)

    _writeback(deg_sh, stage, out0, out1, c, s)


_deg_call = pl.kernel(
    _deg_body,
    out_type=(jax.ShapeDtypeStruct((NP,), F32),
              jax.ShapeDtypeStruct((NP,), F32)),
    mesh=_MESH,
    scratch_types=[
        pltpu.VMEM_SHARED((NP,), F32),
        pltpu.VMEM((BLK,), F32),
        pltpu.VMEM((ROWS_PT,), F32),
        pltpu.SemaphoreType.DMA,
        pltpu.SemaphoreType.DMA,
    ] + [pltpu.VMEM((BLK,), I32)] * CB,
)


def _s1_body(srcs, dsts, g1_h, zeros_h, out0, out1, *scr):
    acc_sh, stage, sem_in, sem_g, sem_s = scr[:5]
    src_b = scr[5:5 + CB]
    dst_b = scr[5 + CB:5 + 2 * CB]
    rows_b = scr[5 + 2 * CB:5 + 3 * CB]
    c = lax.axis_index("c")
    s = lax.axis_index("s")
    wid = c * NS + s
    pltpu.sync_copy(zeros_h.at[pl.ds(s * ROWS_PT, ROWS_PT)], stage)
    pltpu.sync_copy(stage, acc_sh.at[pl.ds(s * ROWS_PT, ROWS_PT)])
    plsc.subcore_barrier()

    def chunk(i, carry):
        @pl.when(i > 0)
        def _():
            for j in range(CB):
                pltpu.make_async_copy(rows_b[j], acc_sh.at[dst_b[j]],
                                      sem_s).wait()
        base = wid * (NB * BLK) + i * (CB * BLK)
        for j in range(CB):
            pltpu.async_copy(srcs.at[pl.ds(base + j * BLK, BLK)], src_b[j],
                             sem_in)
            pltpu.async_copy(dsts.at[pl.ds(base + j * BLK, BLK)], dst_b[j],
                             sem_in)
        for j in range(CB):
            pltpu.make_async_copy(srcs.at[pl.ds(base + j * BLK, BLK)],
                                  src_b[j], sem_in).wait()
            pltpu.make_async_copy(dsts.at[pl.ds(base + j * BLK, BLK)],
                                  dst_b[j], sem_in).wait()
        for j in range(CB):
            pltpu.async_copy(g1_h.at[src_b[j]], rows_b[j], sem_g)
        for j in range(CB):
            pltpu.make_async_copy(g1_h.at[src_b[j]], rows_b[j], sem_g).wait()
        for j in range(CB):
            pltpu.async_copy(rows_b[j], acc_sh.at[dst_b[j]], sem_s, add=True)
        return carry

    lax.fori_loop(0, NCH, chunk, 0)
    for j in range(CB):
        pltpu.make_async_copy(rows_b[j], acc_sh.at[dst_b[j]], sem_s).wait()

    @pl.when(wid < 16)
    def _():
        base = MAIN + wid * BLK
        pltpu.sync_copy(srcs.at[pl.ds(base, BLK)], src_b[0])
        pltpu.sync_copy(dsts.at[pl.ds(base, BLK)], dst_b[0])
        pltpu.sync_copy(g1_h.at[src_b[0]], rows_b[0])
        pltpu.sync_copy(rows_b[0], acc_sh.at[dst_b[0]], add=True)

    _writeback(acc_sh, stage, out0, out1, c, s)


_s1_call = pl.kernel(
    _s1_body,
    out_type=(jax.ShapeDtypeStruct((NP, 8), F32),
              jax.ShapeDtypeStruct((NP, 8), F32)),
    mesh=_MESH,
    scratch_types=[
        pltpu.VMEM_SHARED((NP, 8), F32),
        pltpu.VMEM((ROWS_PT, 8), F32),
        pltpu.SemaphoreType.DMA,
        pltpu.SemaphoreType.DMA,
        pltpu.SemaphoreType.DMA,
    ] + [pltpu.VMEM((BLK,), I32)] * (2 * CB)
      + [pltpu.VMEM((BLK, 8), F32)] * CB,
    compiler_params=pltpu.CompilerParams(use_tc_tiling_on_sc=False),
)


def _s2_body(srcs, dsts, z_h, zeros_h, out0, out1, *scr):
    acc_sh, stage, sem_in, sem_g, sem_s = scr[:5]
    src_b = scr[5:5 + CB]
    dst_b = scr[5 + CB:5 + 2 * CB]
    val_b = scr[5 + 2 * CB:5 + 3 * CB]
    c = lax.axis_index("c")
    s = lax.axis_index("s")
    wid = c * NS + s
    pltpu.sync_copy(zeros_h.at[pl.ds(s * ROWS_PT, ROWS_PT)], stage)
    pltpu.sync_copy(stage, acc_sh.at[pl.ds(s * ROWS_PT, ROWS_PT)])
    plsc.subcore_barrier()

    def chunk(i, carry):
        @pl.when(i > 0)
        def _():
            for j in range(CB):
                pltpu.make_async_copy(val_b[j], acc_sh.at[dst_b[j]],
                                      sem_s).wait()
        base = wid * (NB * BLK) + i * (CB * BLK)
        for j in range(CB):
            pltpu.async_copy(srcs.at[pl.ds(base + j * BLK, BLK)], src_b[j],
                             sem_in)
            pltpu.async_copy(dsts.at[pl.ds(base + j * BLK, BLK)], dst_b[j],
                             sem_in)
        for j in range(CB):
            pltpu.make_async_copy(srcs.at[pl.ds(base + j * BLK, BLK)],
                                  src_b[j], sem_in).wait()
            pltpu.make_async_copy(dsts.at[pl.ds(base + j * BLK, BLK)],
                                  dst_b[j], sem_in).wait()
        for j in range(CB):
            pltpu.async_copy(z_h.at[src_b[j]], val_b[j], sem_g)
        for j in range(CB):
            pltpu.make_async_copy(z_h.at[src_b[j]], val_b[j], sem_g).wait()
        for j in range(CB):
            pltpu.async_copy(val_b[j], acc_sh.at[dst_b[j]], sem_s, add=True)
        return carry

    lax.fori_loop(0, NCH, chunk, 0)
    for j in range(CB):
        pltpu.make_async_copy(val_b[j], acc_sh.at[dst_b[j]], sem_s).wait()

    @pl.when(wid < 16)
    def _():
        base = MAIN + wid * BLK
        pltpu.sync_copy(srcs.at[pl.ds(base, BLK)], src_b[0])
        pltpu.sync_copy(dsts.at[pl.ds(base, BLK)], dst_b[0])
        pltpu.sync_copy(z_h.at[src_b[0]], val_b[0])
        pltpu.sync_copy(val_b[0], acc_sh.at[dst_b[0]], add=True)

    _writeback(acc_sh, stage, out0, out1, c, s)


_s2_call = pl.kernel(
    _s2_body,
    out_type=(jax.ShapeDtypeStruct((NP,), F32),
              jax.ShapeDtypeStruct((NP,), F32)),
    mesh=_MESH,
    scratch_types=[
        pltpu.VMEM_SHARED((NP,), F32),
        pltpu.VMEM((ROWS_PT,), F32),
        pltpu.SemaphoreType.DMA,
        pltpu.SemaphoreType.DMA,
        pltpu.SemaphoreType.DMA,
    ] + [pltpu.VMEM((BLK,), I32)] * (2 * CB)
      + [pltpu.VMEM((BLK,), F32)] * CB,
)


_R = 512
_GRID = NP // _R


def _tc1_body(deg0_ref, deg1_ref, x_ref, dinv_ref, g1_ref):
    deg = deg0_ref[...] + deg1_ref[...] + 1.0
    dinv = lax.rsqrt(deg)
    dinv_ref[...] = dinv
    g1_ref[...] = jnp.concatenate(
        [x_ref[...] * dinv, jnp.zeros((_R, 3), F32)], axis=1)


def _tc1_call(deg0, deg1, x_pad):
    return pl.pallas_call(
        _tc1_body,
        grid=(_GRID,),
        in_specs=[
            pl.BlockSpec((_R, 1), lambda i: (i, 0)),
            pl.BlockSpec((_R, 1), lambda i: (i, 0)),
            pl.BlockSpec((_R, 5), lambda i: (i, 0)),
        ],
        out_specs=[
            pl.BlockSpec((_R, 1), lambda i: (i, 0)),
            pl.BlockSpec((_R, 8), lambda i: (i, 0)),
        ],
        out_shape=[
            jax.ShapeDtypeStruct((NP, 1), F32),
            jax.ShapeDtypeStruct((NP, 8), F32),
        ],
    )(deg0, deg1, x_pad)


def _tc2_body(s1p0_ref, s1p1_ref, x_ref, dinv_ref, w1_ref, b1_ref, w2_ref,
              z_ref):
    s1 = s1p0_ref[...] + s1p1_ref[...]
    dinv = dinv_ref[...]
    u = dinv * s1[:, :5] + (dinv * dinv) * x_ref[...]
    h = jnp.maximum(
        jnp.dot(u, w1_ref[...], preferred_element_type=F32) + b1_ref[...],
        0.0)
    z_ref[...] = dinv * jnp.dot(h, w2_ref[...], preferred_element_type=F32)


def _tc2_call(s1p0, s1p1, x_pad, dinv, W1, b1r, W2):
    return pl.pallas_call(
        _tc2_body,
        grid=(_GRID,),
        in_specs=[
            pl.BlockSpec((_R, 8), lambda i: (i, 0)),
            pl.BlockSpec((_R, 8), lambda i: (i, 0)),
            pl.BlockSpec((_R, 5), lambda i: (i, 0)),
            pl.BlockSpec((_R, 1), lambda i: (i, 0)),
            pl.BlockSpec((5, 16), lambda i: (0, 0)),
            pl.BlockSpec((1, 16), lambda i: (0, 0)),
            pl.BlockSpec((16, 1), lambda i: (0, 0)),
        ],
        out_specs=pl.BlockSpec((_R, 1), lambda i: (i, 0)),
        out_shape=jax.ShapeDtypeStruct((NP, 1), F32),
    )(s1p0, s1p1, x_pad, dinv, W1, b1r, W2)


def _tc3_body(s2p0_ref, s2p1_ref, dinv_ref, z_ref, b2_ref, out_ref):
    dinv = dinv_ref[...]
    out_ref[...] = dinv * (s2p0_ref[...] + s2p1_ref[...]) \
        + dinv * z_ref[...] + b2_ref[...]


def _tc3_call(s2p0, s2p1, dinv, z, b2r):
    return pl.pallas_call(
        _tc3_body,
        grid=(_GRID,),
        in_specs=[
            pl.BlockSpec((_R, 1), lambda i: (i, 0)),
            pl.BlockSpec((_R, 1), lambda i: (i, 0)),
            pl.BlockSpec((_R, 1), lambda i: (i, 0)),
            pl.BlockSpec((_R, 1), lambda i: (i, 0)),
            pl.BlockSpec((1, 1), lambda i: (0, 0)),
        ],
        out_specs=pl.BlockSpec((_R, 1), lambda i: (i, 0)),
        out_shape=jax.ShapeDtypeStruct((NP, 1), F32),
    )(s2p0, s2p1, dinv, z, b2r)


def kernel(x, edge_index, W1, b1, W2, b2):
    srcs = edge_index[0]
    dsts = edge_index[1]
    x_pad = jnp.pad(x, ((0, NP - N_NODES), (0, 0)))
    zeros1 = jnp.zeros((NP,), F32)
    zeros8 = jnp.zeros((NP, 8), F32)

    deg0, deg1 = _deg_call(dsts, zeros1)
    dinv, g1 = _tc1_call(deg0.reshape(NP, 1),
                         deg1.reshape(NP, 1), x_pad)
    s1p0, s1p1 = _s1_call(srcs, dsts, g1, zeros8)
    z = _tc2_call(s1p0, s1p1, x_pad, dinv,
                  W1, b1.reshape(1, 16), W2)
    s2p0, s2p1 = _s2_call(srcs, dsts, z.reshape(NP), zeros1)
    out = _tc3_call(s2p0.reshape(NP, 1), s2p1.reshape(NP, 1),
                    dinv, z, b2.reshape(1, 1))
    return out[:N_NODES]


# speedup vs baseline: 160.0262x; 1.4858x over previous
import jax
import jax.numpy as jnp
from jax import lax
from jax.experimental import pallas as pl
from jax.experimental.pallas import tpu as pltpu
from jax.experimental.pallas import tpu_sc as plsc

N_NODES = 100000
N_EDGES = 6400000
NP = 100352
NC, NS = 2, 16
NW = NC * NS
BLK = 128
CB = 22
NCH = 71
NB = NCH * CB
MAIN = NW * NB * BLK
ROWS_PT = NP // NS

_MESH = plsc.VectorSubcoreMesh(
    core_axis_name="c", subcore_axis_name="s", num_cores=NC, num_subcores=NS)

F32 = jnp.float32
I32 = jnp.int32


def _writeback(acc_sh, stage, out0, out1, c, s):
    plsc.subcore_barrier()
    pltpu.sync_copy(acc_sh.at[pl.ds(s * ROWS_PT, ROWS_PT)], stage)

    @pl.when(c == 0)
    def _():
        pltpu.sync_copy(stage, out0.at[pl.ds(s * ROWS_PT, ROWS_PT)])

    @pl.when(c == 1)
    def _():
        pltpu.sync_copy(stage, out1.at[pl.ds(s * ROWS_PT, ROWS_PT)])


def _deg_body(dsts, zeros_h, out0, out1, *scr):
    deg_sh, ones_v, stage, sem_in, sem_s = scr[:5]
    dst_b = scr[5:5 + CB]
    c = lax.axis_index("c")
    s = lax.axis_index("s")
    wid = c * NS + s
    one = jnp.ones((16,), F32)
    for i in range(BLK // 16):
        ones_v[pl.ds(i * 16, 16)] = one
    pltpu.sync_copy(zeros_h.at[pl.ds(s * ROWS_PT, ROWS_PT)], stage)
    pltpu.sync_copy(stage, deg_sh.at[pl.ds(s * ROWS_PT, ROWS_PT)])
    plsc.subcore_barrier()

    def chunk(i, carry):
        @pl.when(i > 0)
        def _():
            for j in range(CB):
                pltpu.make_async_copy(ones_v, deg_sh.at[dst_b[j]],
                                      sem_s).wait()
        base = wid * (NB * BLK) + i * (CB * BLK)
        for j in range(CB):
            pltpu.async_copy(dsts.at[pl.ds(base + j * BLK, BLK)], dst_b[j],
                             sem_in)
        for j in range(CB):
            pltpu.make_async_copy(dsts.at[pl.ds(base + j * BLK, BLK)],
                                  dst_b[j], sem_in).wait()
        for j in range(CB):
            pltpu.async_copy(ones_v, deg_sh.at[dst_b[j]], sem_s, add=True)
        return carry

    lax.fori_loop(0, NCH, chunk, 0)
    for j in range(CB):
        pltpu.make_async_copy(ones_v, deg_sh.at[dst_b[j]], sem_s).wait()

    @pl.when(wid < 16)
    def _():
        base = MAIN + wid * BLK
        pltpu.sync_copy(dsts.at[pl.ds(base, BLK)], dst_b[0])
        pltpu.sync_copy(ones_v, deg_sh.at[dst_b[0]], add=True)

    _writeback(deg_sh, stage, out0, out1, c, s)


_deg_call = pl.kernel(
    _deg_body,
    out_type=(jax.ShapeDtypeStruct((NP,), F32),
              jax.ShapeDtypeStruct((NP,), F32)),
    mesh=_MESH,
    scratch_types=[
        pltpu.VMEM_SHARED((NP,), F32),
        pltpu.VMEM((BLK,), F32),
        pltpu.VMEM((ROWS_PT,), F32),
        pltpu.SemaphoreType.DMA,
        pltpu.SemaphoreType.DMA,
    ] + [pltpu.VMEM((BLK,), I32)] * CB,
)


def _s1_body(srcs, dsts, g1_h, zeros_h, out0, out1, *scr):
    acc_sh, stage, sem_in, sem_g, sem_s = scr[:5]
    src_b = scr[5:5 + CB]
    dst_b = scr[5 + CB:5 + 2 * CB]
    rows_b = scr[5 + 2 * CB:5 + 3 * CB]
    c = lax.axis_index("c")
    s = lax.axis_index("s")
    wid = c * NS + s
    pltpu.sync_copy(zeros_h.at[pl.ds(s * ROWS_PT, ROWS_PT)], stage)
    pltpu.sync_copy(stage, acc_sh.at[pl.ds(s * ROWS_PT, ROWS_PT)])
    plsc.subcore_barrier()

    def chunk(i, carry):
        @pl.when(i > 0)
        def _():
            for j in range(CB):
                pltpu.make_async_copy(rows_b[j], acc_sh.at[dst_b[j]],
                                      sem_s).wait()
        base = wid * (NB * BLK) + i * (CB * BLK)
        for j in range(CB):
            pltpu.async_copy(srcs.at[pl.ds(base + j * BLK, BLK)], src_b[j],
                             sem_in)
            pltpu.async_copy(dsts.at[pl.ds(base + j * BLK, BLK)], dst_b[j],
                             sem_in)
        for j in range(CB):
            pltpu.make_async_copy(srcs.at[pl.ds(base + j * BLK, BLK)],
                                  src_b[j], sem_in).wait()
            pltpu.make_async_copy(dsts.at[pl.ds(base + j * BLK, BLK)],
                                  dst_b[j], sem_in).wait()
        for j in range(CB):
            pltpu.async_copy(g1_h.at[src_b[j]], rows_b[j], sem_g)
        for j in range(CB):
            pltpu.make_async_copy(g1_h.at[src_b[j]], rows_b[j], sem_g).wait()
        for j in range(CB):
            pltpu.async_copy(rows_b[j], acc_sh.at[dst_b[j]], sem_s, add=True)
        return carry

    lax.fori_loop(0, NCH, chunk, 0)
    for j in range(CB):
        pltpu.make_async_copy(rows_b[j], acc_sh.at[dst_b[j]], sem_s).wait()

    @pl.when(wid < 16)
    def _():
        base = MAIN + wid * BLK
        pltpu.sync_copy(srcs.at[pl.ds(base, BLK)], src_b[0])
        pltpu.sync_copy(dsts.at[pl.ds(base, BLK)], dst_b[0])
        pltpu.sync_copy(g1_h.at[src_b[0]], rows_b[0])
        pltpu.sync_copy(rows_b[0], acc_sh.at[dst_b[0]], add=True)

    _writeback(acc_sh, stage, out0, out1, c, s)


_s1_call = pl.kernel(
    _s1_body,
    out_type=(jax.ShapeDtypeStruct((NP, 8), F32),
              jax.ShapeDtypeStruct((NP, 8), F32)),
    mesh=_MESH,
    scratch_types=[
        pltpu.VMEM_SHARED((NP, 8), F32),
        pltpu.VMEM((ROWS_PT, 8), F32),
        pltpu.SemaphoreType.DMA,
        pltpu.SemaphoreType.DMA,
        pltpu.SemaphoreType.DMA,
    ] + [pltpu.VMEM((BLK,), I32)] * (2 * CB)
      + [pltpu.VMEM((BLK, 8), F32)] * CB,
    compiler_params=pltpu.CompilerParams(use_tc_tiling_on_sc=False),
)


def _s2_body(srcs, dsts, z_h, zeros_h, out0, out1, *scr):
    acc_sh, stage, sem_in, sem_g, sem_s = scr[:5]
    src_b = scr[5:5 + CB]
    dst_b = scr[5 + CB:5 + 2 * CB]
    val_b = scr[5 + 2 * CB:5 + 3 * CB]
    c = lax.axis_index("c")
    s = lax.axis_index("s")
    wid = c * NS + s
    pltpu.sync_copy(zeros_h.at[pl.ds(s * ROWS_PT, ROWS_PT)], stage)
    pltpu.sync_copy(stage, acc_sh.at[pl.ds(s * ROWS_PT, ROWS_PT)])
    plsc.subcore_barrier()

    def chunk(i, carry):
        @pl.when(i > 0)
        def _():
            for j in range(CB):
                pltpu.make_async_copy(val_b[j], acc_sh.at[dst_b[j]],
                                      sem_s).wait()
        base = wid * (NB * BLK) + i * (CB * BLK)
        for j in range(CB):
            pltpu.async_copy(srcs.at[pl.ds(base + j * BLK, BLK)], src_b[j],
                             sem_in)
            pltpu.async_copy(dsts.at[pl.ds(base + j * BLK, BLK)], dst_b[j],
                             sem_in)
        for j in range(CB):
            pltpu.make_async_copy(srcs.at[pl.ds(base + j * BLK, BLK)],
                                  src_b[j], sem_in).wait()
            pltpu.make_async_copy(dsts.at[pl.ds(base + j * BLK, BLK)],
                                  dst_b[j], sem_in).wait()
        for j in range(CB):
            pltpu.async_copy(z_h.at[src_b[j]], val_b[j], sem_g)
        for j in range(CB):
            pltpu.make_async_copy(z_h.at[src_b[j]], val_b[j], sem_g).wait()
        for j in range(CB):
            pltpu.async_copy(val_b[j], acc_sh.at[dst_b[j]], sem_s, add=True)
        return carry

    lax.fori_loop(0, NCH, chunk, 0)
    for j in range(CB):
        pltpu.make_async_copy(val_b[j], acc_sh.at[dst_b[j]], sem_s).wait()

    @pl.when(wid < 16)
    def _():
        base = MAIN + wid * BLK
        pltpu.sync_copy(srcs.at[pl.ds(base, BLK)], src_b[0])
        pltpu.sync_copy(dsts.at[pl.ds(base, BLK)], dst_b[0])
        pltpu.sync_copy(z_h.at[src_b[0]], val_b[0])
        pltpu.sync_copy(val_b[0], acc_sh.at[dst_b[0]], add=True)

    _writeback(acc_sh, stage, out0, out1, c, s)


_s2_call = pl.kernel(
    _s2_body,
    out_type=(jax.ShapeDtypeStruct((NP,), F32),
              jax.ShapeDtypeStruct((NP,), F32)),
    mesh=_MESH,
    scratch_types=[
        pltpu.VMEM_SHARED((NP,), F32),
        pltpu.VMEM((ROWS_PT,), F32),
        pltpu.SemaphoreType.DMA,
        pltpu.SemaphoreType.DMA,
        pltpu.SemaphoreType.DMA,
    ] + [pltpu.VMEM((BLK,), I32)] * (2 * CB)
      + [pltpu.VMEM((BLK,), F32)] * CB,
)


NR = NP // 128


def _tc1_body(deg0_ref, deg1_ref, xt_ref, dinv_ref, g1t_ref):
    deg = deg0_ref[...] + deg1_ref[...] + 1.0
    dinv = lax.rsqrt(deg)
    dinv_ref[...] = dinv
    for k in range(5):
        g1t_ref[k] = dinv * xt_ref[k]
    zero = jnp.zeros((NR, 128), F32)
    for k in range(5, 8):
        g1t_ref[k] = zero


def _tc1_call(deg0r, deg1r, xt):
    return pl.pallas_call(
        _tc1_body,
        out_shape=[
            jax.ShapeDtypeStruct((NR, 128), F32),
            jax.ShapeDtypeStruct((8, NR, 128), F32),
        ],
    )(deg0r, deg1r, xt)


def _tc2_body(s1t0_ref, s1t1_ref, xt_ref, dinv_ref, w1_ref, b1_ref, w2_ref,
              z_ref):
    dinv = dinv_ref[...]
    dinv2 = dinv * dinv
    u = [dinv * (s1t0_ref[k] + s1t1_ref[k]) + dinv2 * xt_ref[k]
         for k in range(5)]
    acc = jnp.zeros((NR, 128), F32)
    for j in range(16):
        h = b1_ref[0, j]
        for k in range(5):
            h = h + u[k] * w1_ref[k, j]
        acc = acc + jnp.maximum(h, 0.0) * w2_ref[j, 0]
    z_ref[...] = dinv * acc


def _tc2_call(s1t0, s1t1, xt, dinvr, W1, b1r, W2):
    return pl.pallas_call(
        _tc2_body,
        in_specs=[
            pl.BlockSpec(memory_space=pltpu.VMEM),
            pl.BlockSpec(memory_space=pltpu.VMEM),
            pl.BlockSpec(memory_space=pltpu.VMEM),
            pl.BlockSpec(memory_space=pltpu.VMEM),
            pl.BlockSpec(memory_space=pltpu.SMEM),
            pl.BlockSpec(memory_space=pltpu.SMEM),
            pl.BlockSpec(memory_space=pltpu.SMEM),
        ],
        out_shape=jax.ShapeDtypeStruct((NR, 128), F32),
    )(s1t0, s1t1, xt, dinvr, W1, b1r, W2)


def _tc3_body(s2p0_ref, s2p1_ref, dinv_ref, z_ref, b2_ref, out_ref):
    dinv = dinv_ref[...]
    out_ref[...] = dinv * (s2p0_ref[...] + s2p1_ref[...] + z_ref[...]) \
        + b2_ref[0, 0]


def _tc3_call(s2p0r, s2p1r, dinvr, zr, b2r):
    return pl.pallas_call(
        _tc3_body,
        in_specs=[
            pl.BlockSpec(memory_space=pltpu.VMEM),
            pl.BlockSpec(memory_space=pltpu.VMEM),
            pl.BlockSpec(memory_space=pltpu.VMEM),
            pl.BlockSpec(memory_space=pltpu.VMEM),
            pl.BlockSpec(memory_space=pltpu.SMEM),
        ],
        out_shape=jax.ShapeDtypeStruct((NR, 128), F32),
    )(s2p0r, s2p1r, dinvr, zr, b2r)


def kernel(x, edge_index, W1, b1, W2, b2):
    srcs = edge_index[0]
    dsts = edge_index[1]
    x_pad = jnp.pad(x, ((0, NP - N_NODES), (0, 0)))
    xt = jnp.transpose(x_pad).reshape(5, NR, 128)
    zeros1 = jnp.zeros((NP,), F32)
    zeros8 = jnp.zeros((NP, 8), F32)

    deg0, deg1 = _deg_call(dsts, zeros1)
    dinvr, g1t = _tc1_call(deg0.reshape(NR, 128),
                           deg1.reshape(NR, 128), xt)
    g1 = jnp.transpose(g1t.reshape(8, NP))
    s1p0, s1p1 = _s1_call(srcs, dsts, g1, zeros8)
    s1t0 = jnp.transpose(s1p0).reshape(8, NR, 128)
    s1t1 = jnp.transpose(s1p1).reshape(8, NR, 128)
    zr = _tc2_call(s1t0, s1t1, xt, dinvr,
                   W1, b1.reshape(1, 16), W2)
    s2p0, s2p1 = _s2_call(srcs, dsts, zr.reshape(NP), zeros1)
    outr = _tc3_call(s2p0.reshape(NR, 128), s2p1.reshape(NR, 128),
                     dinvr, zr, b2.reshape(1, 1))
    return outr.reshape(NP, 1)[:N_NODES]


# speedup vs baseline: 192.1419x; 1.2007x over previous
import jax
import jax.numpy as jnp
from jax import lax
from jax.experimental import pallas as pl
from jax.experimental.pallas import tpu as pltpu
from jax.experimental.pallas import tpu_sc as plsc

N_NODES = 100000
N_EDGES = 6400000
NP = 100352
NC, NS = 2, 16
NW = NC * NS
BLK = 128
CB = 22
NCH = 71
NB = NCH * CB
MAIN = NW * NB * BLK
ROWS_PT = NP // NS

_MESH = plsc.VectorSubcoreMesh(
    core_axis_name="c", subcore_axis_name="s", num_cores=NC, num_subcores=NS)

F32 = jnp.float32
I32 = jnp.int32


def _writeback(acc_sh, stage, out0, out1, c, s):
    plsc.subcore_barrier()
    pltpu.sync_copy(acc_sh.at[pl.ds(s * ROWS_PT, ROWS_PT)], stage)

    @pl.when(c == 0)
    def _():
        pltpu.sync_copy(stage, out0.at[pl.ds(s * ROWS_PT, ROWS_PT)])

    @pl.when(c == 1)
    def _():
        pltpu.sync_copy(stage, out1.at[pl.ds(s * ROWS_PT, ROWS_PT)])


def _deg_body(dsts, zeros_h, out0, out1, *scr):
    deg_sh, ones_v, stage, sem_in, sem_s = scr[:5]
    dst_b = scr[5:5 + CB]
    c = lax.axis_index("c")
    s = lax.axis_index("s")
    wid = c * NS + s
    one = jnp.ones((16,), F32)
    for i in range(BLK // 16):
        ones_v[pl.ds(i * 16, 16)] = one
    pltpu.sync_copy(zeros_h.at[pl.ds(s * ROWS_PT, ROWS_PT)], stage)
    pltpu.sync_copy(stage, deg_sh.at[pl.ds(s * ROWS_PT, ROWS_PT)])
    plsc.subcore_barrier()

    def chunk(i, carry):
        @pl.when(i > 0)
        def _():
            for j in range(CB):
                pltpu.make_async_copy(ones_v, deg_sh.at[dst_b[j]],
                                      sem_s).wait()
        base = wid * (NB * BLK) + i * (CB * BLK)
        for j in range(CB):
            pltpu.async_copy(dsts.at[pl.ds(base + j * BLK, BLK)], dst_b[j],
                             sem_in)
        for j in range(CB):
            pltpu.make_async_copy(dsts.at[pl.ds(base + j * BLK, BLK)],
                                  dst_b[j], sem_in).wait()
        for j in range(CB):
            pltpu.async_copy(ones_v, deg_sh.at[dst_b[j]], sem_s, add=True)
        return carry

    lax.fori_loop(0, NCH, chunk, 0)
    for j in range(CB):
        pltpu.make_async_copy(ones_v, deg_sh.at[dst_b[j]], sem_s).wait()

    @pl.when(wid < 16)
    def _():
        base = MAIN + wid * BLK
        pltpu.sync_copy(dsts.at[pl.ds(base, BLK)], dst_b[0])
        pltpu.sync_copy(ones_v, deg_sh.at[dst_b[0]], add=True)

    _writeback(deg_sh, stage, out0, out1, c, s)


_deg_call = pl.kernel(
    _deg_body,
    out_type=(jax.ShapeDtypeStruct((NP,), F32),
              jax.ShapeDtypeStruct((NP,), F32)),
    mesh=_MESH,
    scratch_types=[
        pltpu.VMEM_SHARED((NP,), F32),
        pltpu.VMEM((BLK,), F32),
        pltpu.VMEM((ROWS_PT,), F32),
        pltpu.SemaphoreType.DMA,
        pltpu.SemaphoreType.DMA,
    ] + [pltpu.VMEM((BLK,), I32)] * CB,
)


def _s1_body(srcs, dsts, g1_h, zeros_h, out0, out1, *scr):
    acc_sh, stage, sem_in, sem_g, sem_s = scr[:5]
    src_b = scr[5:5 + CB]
    dst_b = scr[5 + CB:5 + 2 * CB]
    rows_b = scr[5 + 2 * CB:5 + 3 * CB]
    c = lax.axis_index("c")
    s = lax.axis_index("s")
    wid = c * NS + s
    pltpu.sync_copy(zeros_h.at[pl.ds(s * ROWS_PT, ROWS_PT)], stage)
    pltpu.sync_copy(stage, acc_sh.at[pl.ds(s * ROWS_PT, ROWS_PT)])
    plsc.subcore_barrier()

    def chunk(i, carry):
        @pl.when(i > 0)
        def _():
            for j in range(CB):
                pltpu.make_async_copy(rows_b[j], acc_sh.at[dst_b[j]],
                                      sem_s).wait()
        base = wid * (NB * BLK) + i * (CB * BLK)
        for j in range(CB):
            pltpu.async_copy(srcs.at[pl.ds(base + j * BLK, BLK)], src_b[j],
                             sem_in)
            pltpu.async_copy(dsts.at[pl.ds(base + j * BLK, BLK)], dst_b[j],
                             sem_in)
        for j in range(CB):
            pltpu.make_async_copy(srcs.at[pl.ds(base + j * BLK, BLK)],
                                  src_b[j], sem_in).wait()
            pltpu.make_async_copy(dsts.at[pl.ds(base + j * BLK, BLK)],
                                  dst_b[j], sem_in).wait()
        for j in range(CB):
            pltpu.async_copy(g1_h.at[src_b[j]], rows_b[j], sem_g)
        for j in range(CB):
            pltpu.make_async_copy(g1_h.at[src_b[j]], rows_b[j], sem_g).wait()
        for j in range(CB):
            pltpu.async_copy(rows_b[j], acc_sh.at[dst_b[j]], sem_s, add=True)
        return carry

    lax.fori_loop(0, NCH, chunk, 0)
    for j in range(CB):
        pltpu.make_async_copy(rows_b[j], acc_sh.at[dst_b[j]], sem_s).wait()

    @pl.when(wid < 16)
    def _():
        base = MAIN + wid * BLK
        pltpu.sync_copy(srcs.at[pl.ds(base, BLK)], src_b[0])
        pltpu.sync_copy(dsts.at[pl.ds(base, BLK)], dst_b[0])
        pltpu.sync_copy(g1_h.at[src_b[0]], rows_b[0])
        pltpu.sync_copy(rows_b[0], acc_sh.at[dst_b[0]], add=True)

    _writeback(acc_sh, stage, out0, out1, c, s)


_s1_call = pl.kernel(
    _s1_body,
    out_type=(jax.ShapeDtypeStruct((NP, 8), F32),
              jax.ShapeDtypeStruct((NP, 8), F32)),
    mesh=_MESH,
    scratch_types=[
        pltpu.VMEM_SHARED((NP, 8), F32),
        pltpu.VMEM((ROWS_PT, 8), F32),
        pltpu.SemaphoreType.DMA,
        pltpu.SemaphoreType.DMA,
        pltpu.SemaphoreType.DMA,
    ] + [pltpu.VMEM((BLK,), I32)] * (2 * CB)
      + [pltpu.VMEM((BLK, 8), F32)] * CB,
    compiler_params=pltpu.CompilerParams(use_tc_tiling_on_sc=False),
)


def _s2_body(srcs, dsts, z_h, zeros_h, out0, out1, *scr):
    acc_sh, z_sh, stage, sem_in, sem_g, sem_s = scr[:6]
    src_b = scr[6:6 + CB]
    dst_b = scr[6 + CB:6 + 2 * CB]
    val_b = scr[6 + 2 * CB:6 + 3 * CB]
    c = lax.axis_index("c")
    s = lax.axis_index("s")
    wid = c * NS + s
    pltpu.sync_copy(zeros_h.at[pl.ds(s * ROWS_PT, ROWS_PT)], stage)
    pltpu.sync_copy(stage, acc_sh.at[pl.ds(s * ROWS_PT, ROWS_PT)])
    pltpu.sync_copy(z_h.at[pl.ds(s * ROWS_PT, ROWS_PT)], stage)
    pltpu.sync_copy(stage, z_sh.at[pl.ds(s * ROWS_PT, ROWS_PT)])
    plsc.subcore_barrier()

    def chunk(i, carry):
        @pl.when(i > 0)
        def _():
            for j in range(CB):
                pltpu.make_async_copy(val_b[j], acc_sh.at[dst_b[j]],
                                      sem_s).wait()
        base = wid * (NB * BLK) + i * (CB * BLK)
        for j in range(CB):
            pltpu.async_copy(srcs.at[pl.ds(base + j * BLK, BLK)], src_b[j],
                             sem_in)
            pltpu.async_copy(dsts.at[pl.ds(base + j * BLK, BLK)], dst_b[j],
                             sem_in)
        for j in range(CB):
            pltpu.make_async_copy(srcs.at[pl.ds(base + j * BLK, BLK)],
                                  src_b[j], sem_in).wait()
            pltpu.make_async_copy(dsts.at[pl.ds(base + j * BLK, BLK)],
                                  dst_b[j], sem_in).wait()
        for j in range(CB):
            pltpu.async_copy(z_sh.at[src_b[j]], val_b[j], sem_g)
        for j in range(CB):
            pltpu.make_async_copy(z_sh.at[src_b[j]], val_b[j],
                                  sem_g).wait()
        for j in range(CB):
            pltpu.async_copy(val_b[j], acc_sh.at[dst_b[j]], sem_s, add=True)
        return carry

    lax.fori_loop(0, NCH, chunk, 0)
    for j in range(CB):
        pltpu.make_async_copy(val_b[j], acc_sh.at[dst_b[j]], sem_s).wait()

    @pl.when(wid < 16)
    def _():
        base = MAIN + wid * BLK
        pltpu.sync_copy(srcs.at[pl.ds(base, BLK)], src_b[0])
        pltpu.sync_copy(dsts.at[pl.ds(base, BLK)], dst_b[0])
        pltpu.sync_copy(z_sh.at[src_b[0]], val_b[0])
        pltpu.sync_copy(val_b[0], acc_sh.at[dst_b[0]], add=True)

    _writeback(acc_sh, stage, out0, out1, c, s)


_s2_call = pl.kernel(
    _s2_body,
    out_type=(jax.ShapeDtypeStruct((NP,), F32),
              jax.ShapeDtypeStruct((NP,), F32)),
    mesh=_MESH,
    scratch_types=[
        pltpu.VMEM_SHARED((NP,), F32),
        pltpu.VMEM_SHARED((NP,), F32),
        pltpu.VMEM((ROWS_PT,), F32),
        pltpu.SemaphoreType.DMA,
        pltpu.SemaphoreType.DMA,
        pltpu.SemaphoreType.DMA,
    ] + [pltpu.VMEM((BLK,), I32)] * (2 * CB)
      + [pltpu.VMEM((BLK,), F32)] * CB,
)


NR = NP // 128


def _tc1_body(deg0_ref, deg1_ref, xt_ref, dinv_ref, g1t_ref):
    deg = deg0_ref[...] + deg1_ref[...] + 1.0
    dinv = lax.rsqrt(deg)
    dinv_ref[...] = dinv
    for k in range(5):
        g1t_ref[k] = dinv * xt_ref[k]
    zero = jnp.zeros((NR, 128), F32)
    for k in range(5, 8):
        g1t_ref[k] = zero


def _tc1_call(deg0r, deg1r, xt):
    return pl.pallas_call(
        _tc1_body,
        out_shape=[
            jax.ShapeDtypeStruct((NR, 128), F32),
            jax.ShapeDtypeStruct((8, NR, 128), F32),
        ],
    )(deg0r, deg1r, xt)


def _tc2_body(s1t0_ref, s1t1_ref, xt_ref, dinv_ref, w1_ref, b1_ref, w2_ref,
              z_ref):
    dinv = dinv_ref[...]
    dinv2 = dinv * dinv
    u = [dinv * (s1t0_ref[k] + s1t1_ref[k]) + dinv2 * xt_ref[k]
         for k in range(5)]
    acc = jnp.zeros((NR, 128), F32)
    for j in range(16):
        h = b1_ref[0, j]
        for k in range(5):
            h = h + u[k] * w1_ref[k, j]
        acc = acc + jnp.maximum(h, 0.0) * w2_ref[j, 0]
    z_ref[...] = dinv * acc


def _tc2_call(s1t0, s1t1, xt, dinvr, W1, b1r, W2):
    return pl.pallas_call(
        _tc2_body,
        in_specs=[
            pl.BlockSpec(memory_space=pltpu.VMEM),
            pl.BlockSpec(memory_space=pltpu.VMEM),
            pl.BlockSpec(memory_space=pltpu.VMEM),
            pl.BlockSpec(memory_space=pltpu.VMEM),
            pl.BlockSpec(memory_space=pltpu.SMEM),
            pl.BlockSpec(memory_space=pltpu.SMEM),
            pl.BlockSpec(memory_space=pltpu.SMEM),
        ],
        out_shape=jax.ShapeDtypeStruct((NR, 128), F32),
    )(s1t0, s1t1, xt, dinvr, W1, b1r, W2)


def _tc3_body(s2p0_ref, s2p1_ref, dinv_ref, z_ref, b2_ref, out_ref):
    dinv = dinv_ref[...]
    out_ref[...] = dinv * (s2p0_ref[...] + s2p1_ref[...] + z_ref[...]) \
        + b2_ref[0, 0]


def _tc3_call(s2p0r, s2p1r, dinvr, zr, b2r):
    return pl.pallas_call(
        _tc3_body,
        in_specs=[
            pl.BlockSpec(memory_space=pltpu.VMEM),
            pl.BlockSpec(memory_space=pltpu.VMEM),
            pl.BlockSpec(memory_space=pltpu.VMEM),
            pl.BlockSpec(memory_space=pltpu.VMEM),
            pl.BlockSpec(memory_space=pltpu.SMEM),
        ],
        out_shape=jax.ShapeDtypeStruct((NR, 128), F32),
    )(s2p0r, s2p1r, dinvr, zr, b2r)


def kernel(x, edge_index, W1, b1, W2, b2):
    srcs = edge_index[0]
    dsts = edge_index[1]
    x_pad = jnp.pad(x, ((0, NP - N_NODES), (0, 0)))
    xt = jnp.transpose(x_pad).reshape(5, NR, 128)
    zeros1 = jnp.zeros((NP,), F32)
    zeros8 = jnp.zeros((NP, 8), F32)

    deg0, deg1 = _deg_call(dsts, zeros1)
    dinvr, g1t = _tc1_call(deg0.reshape(NR, 128),
                           deg1.reshape(NR, 128), xt)
    g1 = jnp.transpose(g1t.reshape(8, NP))
    s1p0, s1p1 = _s1_call(srcs, dsts, g1, zeros8)
    s1t0 = jnp.transpose(s1p0).reshape(8, NR, 128)
    s1t1 = jnp.transpose(s1p1).reshape(8, NR, 128)
    zr = _tc2_call(s1t0, s1t1, xt, dinvr,
                   W1, b1.reshape(1, 16), W2)
    s2p0, s2p1 = _s2_call(srcs, dsts, zr.reshape(NP), zeros1)
    outr = _tc3_call(s2p0.reshape(NR, 128), s2p1.reshape(NR, 128),
                     dinvr, zr, b2.reshape(1, 1))
    return outr.reshape(NP, 1)[:N_NODES]
